# Initial kernel scaffold; baseline (speedup 1.0000x reference)
import jax
import jax.numpy as jnp
from jax.experimental import pallas as pl


def kernel(x, edge_index, W1_l, b1, W1_r, W2_l, b2, W2_r):
    raise NotImplementedError("write your pallas kernel here")


# speedup vs baseline: 2.7794x; 2.7794x over previous
import functools

import jax
import jax.numpy as jnp
from jax import lax
from jax.experimental import pallas as pl
from jax.experimental.pallas import tpu as pltpu
from jax.experimental.pallas import tpu_sc as plsc

_N = 10000
_E = 320000
_D = 128

_NC = 2
_NS = 16
_NW = _NC * _NS
_K = 128
_NCHUNK = 80
_EPT = _K * _NCHUNK
_EPAD = _NW * _EPT
_NP = 10112
_PAD_DST = 10008
_RPT = _NP // _NS
_CW = 128

_MESH = plsc.VectorSubcoreMesh(core_axis_name="c", subcore_axis_name="s",
                               num_cores=_NC, num_subcores=_NS)


def _agg_body(y_hbm, src_hbm, dst_hbm, z_hbm, out_hbm,
              acc, src1, dst1, rows, sem):
    c = lax.axis_index("c")
    s = lax.axis_index("s")
    tid = c * _NS + s
    r0 = s * _RPT

    pltpu.sync_copy(z_hbm.at[pl.ds(r0, _RPT)], acc.at[pl.ds(r0, _RPT)])
    plsc.subcore_barrier()
    e0 = tid * _EPT

    @pl.loop(0, _NCHUNK)
    def _(j):
        base = e0 + j * _K
        pltpu.sync_copy(src_hbm.at[pl.ds(base, _K)], src1)
        pltpu.sync_copy(dst_hbm.at[pl.ds(base, _K)], dst1)
        pltpu.async_copy(y_hbm.at[src1], rows, sem).wait()
        pltpu.sync_copy(rows, acc.at[dst1], add=True)

    plsc.subcore_barrier()
    pltpu.sync_copy(acc.at[pl.ds(r0, _RPT)],
                    out_hbm.at[c].at[pl.ds(r0, _RPT)])


_agg = pl.kernel(
    _agg_body,
    out_type=jax.ShapeDtypeStruct((_NC, _NP, _D), jnp.float32),
    mesh=_MESH,
    scratch_types=[
        pltpu.VMEM_SHARED((_NP, _D), jnp.float32),
        pltpu.VMEM((_K,), jnp.int32),
        pltpu.VMEM((_K,), jnp.int32),
        pltpu.VMEM((_K, _D), jnp.float32),
        pltpu.SemaphoreType.DMA,
    ],
)


def _cnt_body(dst_hbm, zc_hbm, ones_hbm, cnt_hbm, cacc, dst1, onesv):
    c = lax.axis_index("c")
    s = lax.axis_index("s")
    tid = c * _NS + s
    r0 = s * _RPT

    pltpu.sync_copy(zc_hbm.at[pl.ds(r0, _RPT)], cacc.at[pl.ds(r0, _RPT)])
    pltpu.sync_copy(ones_hbm, onesv)
    plsc.subcore_barrier()
    e0 = tid * _EPT

    @pl.loop(0, _NCHUNK)
    def _(j):
        pltpu.sync_copy(dst_hbm.at[pl.ds(e0 + j * _K, _K)], dst1)
        pltpu.sync_copy(onesv, cacc.at[dst1], add=True)

    plsc.subcore_barrier()
    pltpu.sync_copy(cacc.at[pl.ds(r0, _RPT)],
                    cnt_hbm.at[c].at[pl.ds(r0, _RPT)])


_cnt = pl.kernel(
    _cnt_body,
    out_type=jax.ShapeDtypeStruct((_NC, _NP, _CW), jnp.float32),
    mesh=_MESH,
    scratch_types=[
        pltpu.VMEM_SHARED((_NP, _CW), jnp.float32),
        pltpu.VMEM((_K,), jnp.int32),
        pltpu.VMEM((_K, _CW), jnp.float32),
    ],
)


def _lin_body(x_ref, w_ref, b_ref, o_ref):
    o_ref[...] = (
        jnp.dot(x_ref[...], w_ref[...],
                preferred_element_type=jnp.float32,
                precision=lax.Precision.HIGHEST)
        + b_ref[...]
    )


def _lin(x, w_t, b):
    r = 1000
    return pl.pallas_call(
        _lin_body,
        grid=(_N // r,),
        in_specs=[
            pl.BlockSpec((r, _D), lambda i: (i, 0)),
            pl.BlockSpec((_D, _D), lambda i: (0, 0)),
            pl.BlockSpec((1, _D), lambda i: (0, 0)),
        ],
        out_specs=pl.BlockSpec((r, _D), lambda i: (i, 0)),
        out_shape=jax.ShapeDtypeStruct((_N, _D), jnp.float32),
    )(x, w_t, b)


def _combine_body(relu, s0_ref, s1_ref, c0_ref, c1_ref, xr_ref, w_ref, o_ref):
    cnt = c0_ref[...][:, :1] + c1_ref[...][:, :1]
    mean = (s0_ref[...] + s1_ref[...]) / jnp.maximum(cnt, 1.0)
    out = (
        jnp.dot(mean, w_ref[...],
                preferred_element_type=jnp.float32,
                precision=lax.Precision.HIGHEST)
        + xr_ref[...]
    )
    if relu:
        out = jnp.maximum(out, 0.0)
    o_ref[...] = out


def _combine(s, cnt, xr, w_t, relu):
    r = 1000
    return pl.pallas_call(
        functools.partial(_combine_body, relu),
        grid=(_N // r,),
        in_specs=[
            pl.BlockSpec((r, _D), lambda i: (i, 0)),
            pl.BlockSpec((r, _D), lambda i: (i, 0)),
            pl.BlockSpec((r, _CW), lambda i: (i, 0)),
            pl.BlockSpec((r, _CW), lambda i: (i, 0)),
            pl.BlockSpec((r, _D), lambda i: (i, 0)),
            pl.BlockSpec((_D, _D), lambda i: (0, 0)),
        ],
        out_specs=pl.BlockSpec((r, _D), lambda i: (i, 0)),
        out_shape=jax.ShapeDtypeStruct((_N, _D), jnp.float32),
    )(s[0], s[1], cnt[0], cnt[1], xr, w_t)


def kernel(x, edge_index, W1_l, b1, W1_r, W2_l, b2, W2_r):
    npad = _EPAD - _E
    src = jnp.concatenate(
        [edge_index[0].astype(jnp.int32), jnp.zeros((npad,), jnp.int32)])
    dst = jnp.concatenate(
        [edge_index[1].astype(jnp.int32),
         jnp.full((npad,), _PAD_DST, jnp.int32)])
    zeros_acc = jnp.zeros((_NP, _D), jnp.float32)
    zeros_cnt = jnp.zeros((_NP, _CW), jnp.float32)
    ones = jnp.ones((_K, _CW), jnp.float32)

    cnt = _cnt(dst, zeros_cnt, ones)
    s1 = _agg(x, src, dst, zeros_acc)
    xr1 = _lin(x, W1_r.T, b1.reshape(1, _D))
    h = _combine(s1, cnt, xr1, W1_l.T, relu=True)

    s2 = _agg(h, src, dst, zeros_acc)
    xr2 = _lin(h, W2_r.T, b2.reshape(1, _D))
    out = _combine(s2, cnt, xr2, W2_l.T, relu=False)
    return out


# speedup vs baseline: 3.4861x; 1.2543x over previous
import functools

import jax
import jax.numpy as jnp
from jax import lax
from jax.experimental import pallas as pl
from jax.experimental.pallas import tpu as pltpu
from jax.experimental.pallas import tpu_sc as plsc

_N = 10000
_E = 320000
_D = 128

_NC = 2
_NS = 16
_NW = _NC * _NS
_K = 128
_NCHUNK = 80
_HALF = _NCHUNK // 2
_EPT = _K * _NCHUNK
_EPAD = _NW * _EPT
_NP = 10112
_PAD_DST = 10008
_RPT = _NP // _NS
_CW = 128

_MESH = plsc.VectorSubcoreMesh(core_axis_name="c", subcore_axis_name="s",
                               num_cores=_NC, num_subcores=_NS)


def _agg_body(y_hbm, src_hbm, dst_hbm, z_hbm, out_hbm,
              acc, srcv, dstv, rows_a, rows_b, sem_a, sem_b):
    c = lax.axis_index("c")
    s = lax.axis_index("s")
    tid = c * _NS + s
    r0 = s * _RPT

    pltpu.sync_copy(z_hbm.at[pl.ds(r0, _RPT)], acc.at[pl.ds(r0, _RPT)])
    plsc.subcore_barrier()

    def gather(j, rows, sem):
        pltpu.async_copy(y_hbm.at[srcv.at[j]], rows, sem)

    def drain_scatter(j, rows, sem):
        pltpu.make_async_copy(y_hbm.at[srcv.at[j]], rows, sem).wait()
        pltpu.sync_copy(rows, acc.at[dstv.at[j]], add=True)

    for half in range(2):
        i0 = tid * _NCHUNK + half * _HALF
        pltpu.sync_copy(src_hbm.at[pl.ds(i0, _HALF)], srcv)
        pltpu.sync_copy(dst_hbm.at[pl.ds(i0, _HALF)], dstv)

        gather(0, rows_a, sem_a)

        @pl.loop(0, _HALF // 2 - 1)
        def _(j2):
            ja = 2 * j2
            gather(ja + 1, rows_b, sem_b)
            drain_scatter(ja, rows_a, sem_a)
            gather(ja + 2, rows_a, sem_a)
            drain_scatter(ja + 1, rows_b, sem_b)

        gather(_HALF - 1, rows_b, sem_b)
        drain_scatter(_HALF - 2, rows_a, sem_a)
        drain_scatter(_HALF - 1, rows_b, sem_b)

    plsc.subcore_barrier()
    pltpu.sync_copy(acc.at[pl.ds(r0, _RPT)],
                    out_hbm.at[c].at[pl.ds(r0, _RPT)])


_agg = pl.kernel(
    _agg_body,
    out_type=jax.ShapeDtypeStruct((_NC, _NP, _D), jnp.float32),
    mesh=_MESH,
    scratch_types=[
        pltpu.VMEM_SHARED((_NP, _D), jnp.float32),
        pltpu.VMEM((_HALF, _K), jnp.int32),
        pltpu.VMEM((_HALF, _K), jnp.int32),
        pltpu.VMEM((_K, _D), jnp.float32),
        pltpu.VMEM((_K, _D), jnp.float32),
        pltpu.SemaphoreType.DMA,
        pltpu.SemaphoreType.DMA,
    ],
)


def _cnt_body(dst_hbm, zc_hbm, ones_hbm, cnt_hbm, cacc, dstv, onesv):
    c = lax.axis_index("c")
    s = lax.axis_index("s")
    tid = c * _NS + s
    r0 = s * _RPT

    pltpu.sync_copy(zc_hbm.at[pl.ds(r0, _RPT)], cacc.at[pl.ds(r0, _RPT)])
    pltpu.sync_copy(ones_hbm, onesv)
    pltpu.sync_copy(dst_hbm.at[pl.ds(tid * _NCHUNK, _NCHUNK)], dstv)
    plsc.subcore_barrier()

    @pl.loop(0, _NCHUNK)
    def _(j):
        pltpu.sync_copy(onesv, cacc.at[dstv.at[j]], add=True)

    plsc.subcore_barrier()
    pltpu.sync_copy(cacc.at[pl.ds(r0, _RPT)],
                    cnt_hbm.at[c].at[pl.ds(r0, _RPT)])


_cnt = pl.kernel(
    _cnt_body,
    out_type=jax.ShapeDtypeStruct((_NC, _NP, _CW), jnp.float32),
    mesh=_MESH,
    scratch_types=[
        pltpu.VMEM_SHARED((_NP, _CW), jnp.float32),
        pltpu.VMEM((_NCHUNK, _K), jnp.int32),
        pltpu.VMEM((_K, _CW), jnp.float32),
    ],
)


def _lin_body(x_ref, w_ref, b_ref, o_ref):
    o_ref[...] = (
        jnp.dot(x_ref[...], w_ref[...],
                preferred_element_type=jnp.float32,
                precision=lax.Precision.HIGHEST)
        + b_ref[...]
    )


def _lin(x, w_t, b):
    r = 1000
    return pl.pallas_call(
        _lin_body,
        grid=(_N // r,),
        in_specs=[
            pl.BlockSpec((r, _D), lambda i: (i, 0)),
            pl.BlockSpec((_D, _D), lambda i: (0, 0)),
            pl.BlockSpec((1, _D), lambda i: (0, 0)),
        ],
        out_specs=pl.BlockSpec((r, _D), lambda i: (i, 0)),
        out_shape=jax.ShapeDtypeStruct((_N, _D), jnp.float32),
    )(x, w_t, b)


def _combine_body(relu, s0_ref, s1_ref, c0_ref, c1_ref, xr_ref, w_ref, o_ref):
    cnt = c0_ref[...][:, :1] + c1_ref[...][:, :1]
    mean = (s0_ref[...] + s1_ref[...]) / jnp.maximum(cnt, 1.0)
    out = (
        jnp.dot(mean, w_ref[...],
                preferred_element_type=jnp.float32,
                precision=lax.Precision.HIGHEST)
        + xr_ref[...]
    )
    if relu:
        out = jnp.maximum(out, 0.0)
    o_ref[...] = out


def _combine(s, cnt, xr, w_t, relu):
    r = 1000
    return pl.pallas_call(
        functools.partial(_combine_body, relu),
        grid=(_N // r,),
        in_specs=[
            pl.BlockSpec((r, _D), lambda i: (i, 0)),
            pl.BlockSpec((r, _D), lambda i: (i, 0)),
            pl.BlockSpec((r, _CW), lambda i: (i, 0)),
            pl.BlockSpec((r, _CW), lambda i: (i, 0)),
            pl.BlockSpec((r, _D), lambda i: (i, 0)),
            pl.BlockSpec((_D, _D), lambda i: (0, 0)),
        ],
        out_specs=pl.BlockSpec((r, _D), lambda i: (i, 0)),
        out_shape=jax.ShapeDtypeStruct((_N, _D), jnp.float32),
    )(s[0], s[1], cnt[0], cnt[1], xr, w_t)


def kernel(x, edge_index, W1_l, b1, W1_r, W2_l, b2, W2_r):
    npad = _EPAD - _E
    src = jnp.concatenate(
        [edge_index[0].astype(jnp.int32),
         jnp.zeros((npad,), jnp.int32)]).reshape(_EPAD // _K, _K)
    dst = jnp.concatenate(
        [edge_index[1].astype(jnp.int32),
         jnp.full((npad,), _PAD_DST, jnp.int32)]).reshape(_EPAD // _K, _K)
    zeros_acc = jnp.zeros((_NP, _D), jnp.float32)
    zeros_cnt = jnp.zeros((_NP, _CW), jnp.float32)
    ones = jnp.ones((_K, _CW), jnp.float32)

    cnt = _cnt(dst, zeros_cnt, ones)
    s1 = _agg(x, src, dst, zeros_acc)
    xr1 = _lin(x, W1_r.T, b1.reshape(1, _D))
    h = _combine(s1, cnt, xr1, W1_l.T, relu=True)

    s2 = _agg(h, src, dst, zeros_acc)
    xr2 = _lin(h, W2_r.T, b2.reshape(1, _D))
    out = _combine(s2, cnt, xr2, W2_l.T, relu=False)
    return out


# speedup vs baseline: 3.7226x; 1.0678x over previous
import functools

import jax
import jax.numpy as jnp
from jax import lax
from jax.experimental import pallas as pl
from jax.experimental.pallas import tpu as pltpu
from jax.experimental.pallas import tpu_sc as plsc

_N = 10000
_E = 320000
_D = 128

_NC = 2
_NS = 16
_NW = _NC * _NS
_K = 128
_NCHUNK = 80
_WIN = 16
_NWIN0 = 8
_NWIN1 = 2
_EPT = _K * _NCHUNK
_EPAD = _NW * _EPT
_NP = 10112
_PAD_DST = 10008
_RPT = _NP // _NS
_CW = 128

_MESH = plsc.VectorSubcoreMesh(core_axis_name="c", subcore_axis_name="s",
                               num_cores=_NC, num_subcores=_NS)


def _agg_body(y_hbm, src_hbm, dst_hbm, z_hbm, out_hbm,
              acc, srcv, dstv, rows_a, rows_b, sem_a, sem_b):
    c = lax.axis_index("c")
    s = lax.axis_index("s")
    tid = c * _NS + s
    r0 = s * _RPT

    pltpu.sync_copy(z_hbm.at[pl.ds(r0, _RPT)], acc.at[pl.ds(r0, _RPT)])
    plsc.subcore_barrier()

    def gather(j, rows, sem):
        pltpu.async_copy(y_hbm.at[srcv.at[j]], rows, sem)

    def drain_scatter(j, rows, sem):
        pltpu.make_async_copy(y_hbm.at[srcv.at[j]], rows, sem).wait()
        pltpu.sync_copy(rows, acc.at[dstv.at[j]], add=True)

    nwin = jnp.where(c == 0, _NWIN0, _NWIN1)
    tile_chunk0 = jnp.where(
        c == 0, s * (_NWIN0 * _WIN),
        _NS * _NWIN0 * _WIN + s * (_NWIN1 * _WIN))

    @pl.loop(0, nwin)
    def _(w):
        i0 = tile_chunk0 + w * _WIN
        pltpu.sync_copy(src_hbm.at[pl.ds(i0, _WIN)], srcv)
        pltpu.sync_copy(dst_hbm.at[pl.ds(i0, _WIN)], dstv)

        gather(0, rows_a, sem_a)

        @pl.loop(0, _WIN // 2 - 1)
        def _(j2):
            ja = 2 * j2
            gather(ja + 1, rows_b, sem_b)
            drain_scatter(ja, rows_a, sem_a)
            gather(ja + 2, rows_a, sem_a)
            drain_scatter(ja + 1, rows_b, sem_b)

        gather(_WIN - 1, rows_b, sem_b)
        drain_scatter(_WIN - 2, rows_a, sem_a)
        drain_scatter(_WIN - 1, rows_b, sem_b)

    plsc.subcore_barrier()
    pltpu.sync_copy(acc.at[pl.ds(r0, _RPT)],
                    out_hbm.at[c].at[pl.ds(r0, _RPT)])


_agg = pl.kernel(
    _agg_body,
    out_type=jax.ShapeDtypeStruct((_NC, _NP, _D), jnp.float32),
    mesh=_MESH,
    scratch_types=[
        pltpu.VMEM_SHARED((_NP, _D), jnp.float32),
        pltpu.VMEM((_WIN, _K), jnp.int32),
        pltpu.VMEM((_WIN, _K), jnp.int32),
        pltpu.VMEM((_K, _D), jnp.float32),
        pltpu.VMEM((_K, _D), jnp.float32),
        pltpu.SemaphoreType.DMA,
        pltpu.SemaphoreType.DMA,
    ],
)


def _cnt_body(dst_hbm, zc_hbm, ones_hbm, cnt_hbm, cacc, dstv, onesv):
    c = lax.axis_index("c")
    s = lax.axis_index("s")
    tid = c * _NS + s
    r0 = s * _RPT

    pltpu.sync_copy(zc_hbm.at[pl.ds(r0, _RPT)], cacc.at[pl.ds(r0, _RPT)])
    pltpu.sync_copy(ones_hbm, onesv)
    pltpu.sync_copy(dst_hbm.at[pl.ds(tid * _NCHUNK, _NCHUNK)], dstv)
    plsc.subcore_barrier()

    @pl.loop(0, _NCHUNK)
    def _(j):
        pltpu.sync_copy(onesv, cacc.at[dstv.at[j]], add=True)

    plsc.subcore_barrier()
    pltpu.sync_copy(cacc.at[pl.ds(r0, _RPT)],
                    cnt_hbm.at[c].at[pl.ds(r0, _RPT)])


_cnt = pl.kernel(
    _cnt_body,
    out_type=jax.ShapeDtypeStruct((_NC, _NP, _CW), jnp.float32),
    mesh=_MESH,
    scratch_types=[
        pltpu.VMEM_SHARED((_NP, _CW), jnp.float32),
        pltpu.VMEM((_NCHUNK, _K), jnp.int32),
        pltpu.VMEM((_K, _CW), jnp.float32),
    ],
)


def _lin_body(x_ref, w_ref, b_ref, o_ref):
    o_ref[...] = (
        jnp.dot(x_ref[...], w_ref[...],
                preferred_element_type=jnp.float32,
                precision=lax.Precision.HIGHEST)
        + b_ref[...]
    )


def _lin(x, w_t, b):
    r = 1000
    return pl.pallas_call(
        _lin_body,
        grid=(_N // r,),
        in_specs=[
            pl.BlockSpec((r, _D), lambda i: (i, 0)),
            pl.BlockSpec((_D, _D), lambda i: (0, 0)),
            pl.BlockSpec((1, _D), lambda i: (0, 0)),
        ],
        out_specs=pl.BlockSpec((r, _D), lambda i: (i, 0)),
        out_shape=jax.ShapeDtypeStruct((_N, _D), jnp.float32),
    )(x, w_t, b)


def _combine_body(relu, s0_ref, s1_ref, c0_ref, c1_ref, xr_ref, w_ref, o_ref):
    cnt = c0_ref[...][:, :1] + c1_ref[...][:, :1]
    mean = (s0_ref[...] + s1_ref[...]) / jnp.maximum(cnt, 1.0)
    out = (
        jnp.dot(mean, w_ref[...],
                preferred_element_type=jnp.float32,
                precision=lax.Precision.HIGHEST)
        + xr_ref[...]
    )
    if relu:
        out = jnp.maximum(out, 0.0)
    o_ref[...] = out


def _combine(s, cnt, xr, w_t, relu):
    r = 1000
    return pl.pallas_call(
        functools.partial(_combine_body, relu),
        grid=(_N // r,),
        in_specs=[
            pl.BlockSpec((r, _D), lambda i: (i, 0)),
            pl.BlockSpec((r, _D), lambda i: (i, 0)),
            pl.BlockSpec((r, _CW), lambda i: (i, 0)),
            pl.BlockSpec((r, _CW), lambda i: (i, 0)),
            pl.BlockSpec((r, _D), lambda i: (i, 0)),
            pl.BlockSpec((_D, _D), lambda i: (0, 0)),
        ],
        out_specs=pl.BlockSpec((r, _D), lambda i: (i, 0)),
        out_shape=jax.ShapeDtypeStruct((_N, _D), jnp.float32),
    )(s[0], s[1], cnt[0], cnt[1], xr, w_t)


def kernel(x, edge_index, W1_l, b1, W1_r, W2_l, b2, W2_r):
    npad = _EPAD - _E
    src = jnp.concatenate(
        [edge_index[0].astype(jnp.int32),
         jnp.zeros((npad,), jnp.int32)]).reshape(_EPAD // _K, _K)
    dst = jnp.concatenate(
        [edge_index[1].astype(jnp.int32),
         jnp.full((npad,), _PAD_DST, jnp.int32)]).reshape(_EPAD // _K, _K)
    zeros_acc = jnp.zeros((_NP, _D), jnp.float32)
    zeros_cnt = jnp.zeros((_NP, _CW), jnp.float32)
    ones = jnp.ones((_K, _CW), jnp.float32)

    cnt = _cnt(dst, zeros_cnt, ones)
    s1 = _agg(x, src, dst, zeros_acc)
    xr1 = _lin(x, W1_r.T, b1.reshape(1, _D))
    h = _combine(s1, cnt, xr1, W1_l.T, relu=True)

    s2 = _agg(h, src, dst, zeros_acc)
    xr2 = _lin(h, W2_r.T, b2.reshape(1, _D))
    out = _combine(s2, cnt, xr2, W2_l.T, relu=False)
    return out
